# Initial kernel scaffold; baseline (speedup 1.0000x reference)
import jax
import jax.numpy as jnp
from jax.experimental import pallas as pl


def kernel(inputs, embeddings):
    raise NotImplementedError("write your pallas kernel here")


# speedup vs baseline: 5.5603x; 5.5603x over previous
import jax
import jax.numpy as jnp
from jax.experimental import pallas as pl

_BLOCK_ROWS = 512


def _bcast_body(emb_ref, out_ref):
    out_ref[...] = jnp.broadcast_to(emb_ref[...][None], out_ref.shape)


def kernel(inputs, embeddings):
    batch = inputs.shape[0]
    seq, emb = embeddings.shape
    grid = (seq // _BLOCK_ROWS,)
    return pl.pallas_call(
        _bcast_body,
        grid=grid,
        in_specs=[pl.BlockSpec((_BLOCK_ROWS, emb), lambda s: (s, 0))],
        out_specs=pl.BlockSpec((batch, _BLOCK_ROWS, emb), lambda s: (0, s, 0)),
        out_shape=jax.ShapeDtypeStruct((batch, seq, emb), embeddings.dtype),
    )(embeddings)


# speedup vs baseline: 5.7787x; 1.0393x over previous
import jax
import jax.numpy as jnp
from jax.experimental import pallas as pl

_BLOCK_ROWS = 1024


def _bcast_body(emb_ref, out_ref):
    out_ref[...] = jnp.broadcast_to(emb_ref[...][None], out_ref.shape)


def kernel(inputs, embeddings):
    batch = inputs.shape[0]
    seq, emb = embeddings.shape
    grid = (seq // _BLOCK_ROWS,)
    return pl.pallas_call(
        _bcast_body,
        grid=grid,
        in_specs=[pl.BlockSpec((_BLOCK_ROWS, emb), lambda s: (s, 0))],
        out_specs=pl.BlockSpec((batch, _BLOCK_ROWS, emb), lambda s: (0, s, 0)),
        out_shape=jax.ShapeDtypeStruct((batch, seq, emb), embeddings.dtype),
    )(embeddings)
